# Initial kernel scaffold; baseline (speedup 1.0000x reference)
import jax
import jax.numpy as jnp
from jax.experimental import pallas as pl


def kernel(output, labels):
    raise NotImplementedError("write your pallas kernel here")


# speedup vs baseline: 12.6701x; 12.6701x over previous
import jax
import jax.numpy as jnp
from jax.experimental import pallas as pl
from jax.experimental.pallas import tpu as pltpu

_E = 16 * 32 * 32 * 32 * 3 * 7
_LANES = 128
_ROWS = _E // _LANES
_R = 448
_GRID = _ROWS // _R


def _flat_shift(x, prev_rows, f):
    if f == 0:
        return x
    return jnp.concatenate([prev_rows[:, _LANES - f:], x[:, :_LANES - f]], axis=1)


def _body(out_ref, lab_ref, res_ref,
          acc_pb, acc_nb, acc_np, acc_nn, acc_pc, acc_nc, acc_reg):
    pid = pl.program_id(0)

    @pl.when(pid == 0)
    def _init():
        for a in (acc_pb, acc_nb, acc_np, acc_nn, acc_pc, acc_nc, acc_reg):
            a[...] = jnp.zeros_like(a)

    o = out_ref[...]
    t = lab_ref[...]

    rowi = jax.lax.broadcasted_iota(jnp.int32, (_R, _LANES), 0)
    lane = jax.lax.broadcasted_iota(jnp.int32, (_R, _LANES), 1)
    f_idx = (2 * rowi + lane) % 7
    mask0 = f_idx == 0

    pos_sel = jnp.where(mask0 & (t > 0.5), 1.0, 0.0)
    neg_sel = jnp.where(mask0 & (t < -0.5), 1.0, 0.0)

    a = jnp.abs(o)
    base = jnp.log1p(jnp.exp(-a))
    r = jnp.maximum(o, 0.0)
    bce_pos = base + (a - r)
    bce_neg = base + r
    acc_pb[...] += pos_sel * bce_pos
    acc_nb[...] += neg_sel * bce_neg
    acc_np[...] += pos_sel
    acc_nn[...] += neg_sel
    o_ge = o >= 0.0
    acc_pc[...] += jnp.where(o_ge, pos_sel, 0.0)
    acc_nc[...] += jnp.where(o_ge, 0.0, neg_sel)

    prev = jnp.concatenate([jnp.zeros((1, _LANES), jnp.float32), pos_sel[:-1, :]],
                           axis=0)
    w_full = pos_sel
    for f in range(1, 7):
        w_full = w_full + _flat_shift(pos_sel, prev, f)
    w_reg = w_full - pos_sel

    d = o - t
    ad = jnp.abs(d)
    m = jnp.minimum(ad, 1.0)
    l1 = m * (ad - 0.5 * m)
    acc_reg[...] += w_reg * l1

    @pl.when(pid == _GRID - 1)
    def _final():
        n_pos = jnp.sum(acc_np[...])
        n_neg = jnp.sum(acc_nn[...])
        pb = jnp.sum(acc_pb[...])
        nb = jnp.sum(acc_nb[...])
        pc = jnp.sum(acc_pc[...])
        nc = jnp.sum(acc_nc[...])
        classify = 0.5 * pb / n_pos + 0.5 * nb / n_neg
        ar = acc_reg[...]
        loss = classify
        regs = []
        for f in range(1, 7):
            rs = jnp.sum(jnp.where(f_idx == f, ar, 0.0)) / n_pos
            regs.append(rs)
            loss = loss + rs
        vals = [loss, classify] + regs + [pc, n_pos, nc, n_neg]
        for i, v in enumerate(vals):
            res_ref[i] = v


def kernel(output, labels):
    o2 = output.reshape(_ROWS, _LANES)
    t2 = labels.reshape(_ROWS, _LANES)
    res = pl.pallas_call(
        _body,
        grid=(_GRID,),
        in_specs=[
            pl.BlockSpec((_R, _LANES), lambda i: (i, 0)),
            pl.BlockSpec((_R, _LANES), lambda i: (i, 0)),
        ],
        out_specs=pl.BlockSpec(memory_space=pltpu.SMEM),
        out_shape=jax.ShapeDtypeStruct((12,), jnp.float32),
        scratch_shapes=[pltpu.VMEM((_R, _LANES), jnp.float32)] * 7,
        compiler_params=pltpu.CompilerParams(
            dimension_semantics=("arbitrary",)),
    )(o2, t2)
    return tuple(res[i] for i in range(12))


# speedup vs baseline: 191.9767x; 15.1519x over previous
import jax
import jax.numpy as jnp
from jax.experimental import pallas as pl
from jax.experimental.pallas import tpu as pltpu

_PLANES = 16 * 32 * 3 * 7
_GROUPS = _PLANES // 21
_BG = 8
_BP = 21 * _BG
_GRID = _PLANES // _BP


def _body(out_ref, lab_ref, res_ref, acc_ref):
    pid = pl.program_id(0)

    @pl.when(pid == 0)
    def _init():
        acc_ref[...] = jnp.zeros_like(acc_ref)

    z = jnp.zeros((32, 32), jnp.float32)
    part = [z] * 12

    for g in range(_BG):
        for c in range(3):
            p0 = 21 * g + 7 * c
            conf = lab_ref[p0]
            o0 = out_ref[p0]
            pos = jnp.where(conf > 0.5, 1.0, 0.0)
            neg = jnp.where(conf < -0.5, 1.0, 0.0)
            a = jnp.abs(o0)
            base = jnp.log1p(jnp.exp(-a))
            r = jnp.maximum(o0, 0.0)
            part[0] += pos * (base + (a - r))
            part[1] += neg * (base + r)
            part[2] += pos
            part[3] += neg
            ge = o0 >= 0.0
            part[4] += jnp.where(ge, pos, 0.0)
            part[5] += jnp.where(ge, 0.0, neg)
            for f in range(1, 7):
                d = out_ref[p0 + f] - lab_ref[p0 + f]
                ad = jnp.abs(d)
                m = jnp.minimum(ad, 1.0)
                part[5 + f] += pos * (m * (ad - 0.5 * m))

    for q in range(12):
        acc_ref[q] += part[q]

    @pl.when(pid == _GRID - 1)
    def _final():
        sums = [jnp.sum(acc_ref[q]) for q in range(12)]
        pb, nb, n_pos, n_neg, pc, nc = sums[:6]
        classify = 0.5 * pb / n_pos + 0.5 * nb / n_neg
        regs = [sums[5 + f] / n_pos for f in range(1, 7)]
        loss = classify
        for rv in regs:
            loss = loss + rv
        vals = [loss, classify] + regs + [pc, n_pos, nc, n_neg]
        for i, v in enumerate(vals):
            res_ref[i] = v


def kernel(output, labels):
    o3 = output.transpose(0, 1, 4, 5, 2, 3).reshape(_PLANES, 32, 32)
    t3 = labels.transpose(0, 1, 4, 5, 2, 3).reshape(_PLANES, 32, 32)
    res = pl.pallas_call(
        _body,
        grid=(_GRID,),
        in_specs=[
            pl.BlockSpec((_BP, 32, 32), lambda i: (i, 0, 0)),
            pl.BlockSpec((_BP, 32, 32), lambda i: (i, 0, 0)),
        ],
        out_specs=pl.BlockSpec(memory_space=pltpu.SMEM),
        out_shape=jax.ShapeDtypeStruct((12,), jnp.float32),
        scratch_shapes=[pltpu.VMEM((12, 32, 32), jnp.float32)],
        compiler_params=pltpu.CompilerParams(
            dimension_semantics=("arbitrary",)),
    )(o3, t3)
    return tuple(res[i] for i in range(12))
